# baseline (speedup 1.0000x reference)
import functools

import jax
import jax.numpy as jnp
from jax.experimental import pallas as pl
from jax.experimental.pallas import tpu as pltpu

_BN_EPS = 1e-5


def _fused_gcn_kernel(hf_ref, w_ref, gamma_ref, beta_ref, a_ref, out_ref,
                      ht_scratch):
    @pl.when(pl.program_id(0) == 0)
    def _compute_ht():
        ht = jnp.dot(hf_ref[...], w_ref[...],
                     preferred_element_type=jnp.float32)
        mean = jnp.mean(ht, axis=0, keepdims=True)
        var = jnp.mean(jnp.square(ht - mean), axis=0, keepdims=True)
        inv = jax.lax.rsqrt(var + _BN_EPS)
        htn = (ht - mean) * inv * gamma_ref[...] + beta_ref[...]
        ht_scratch[...] = jnp.maximum(htn, 0.0)

    out_ref[...] = jnp.dot(a_ref[...], ht_scratch[...],
                           preferred_element_type=jnp.float32)


@functools.partial(jax.jit, static_argnames=("block_m", "interpret"))
def _gcn(H, A_normalized, W, bn_gamma, bn_beta, block_m=400, interpret=False):
    batch, n, in_dim = H.shape
    out_dim = W.shape[1]
    hf = H.reshape(batch * n, in_dim)
    gamma = bn_gamma.reshape(1, out_dim)
    beta = bn_beta.reshape(1, out_dim)

    grid = (n // block_m,)
    out = pl.pallas_call(
        _fused_gcn_kernel,
        grid=grid,
        in_specs=[
            pl.BlockSpec((batch * n, in_dim), lambda i: (0, 0)),
            pl.BlockSpec((in_dim, out_dim), lambda i: (0, 0)),
            pl.BlockSpec((1, out_dim), lambda i: (0, 0)),
            pl.BlockSpec((1, out_dim), lambda i: (0, 0)),
            pl.BlockSpec((block_m, n), lambda i: (i, 0)),
        ],
        out_specs=pl.BlockSpec((block_m, out_dim), lambda i: (i, 0)),
        out_shape=jax.ShapeDtypeStruct((n, out_dim), jnp.float32),
        scratch_shapes=[pltpu.VMEM((batch * n, out_dim), jnp.float32)],
        compiler_params=pltpu.CompilerParams(
            vmem_limit_bytes=64 * 1024 * 1024),
        interpret=interpret,
    )(hf, W, gamma, beta, A_normalized)
    return out.reshape(batch, n, out_dim)


def kernel(H, A_normalized, W, bn_gamma, bn_beta):
    return _gcn(H, A_normalized, W, bn_gamma, bn_beta, block_m=200)


# speedup vs baseline: 1.0116x; 1.0116x over previous
import functools

import jax
import jax.numpy as jnp
from jax.experimental import pallas as pl
from jax.experimental.pallas import tpu as pltpu

_BN_EPS = 1e-5


def _fused_gcn_kernel(hf_ref, w_ref, gamma_ref, beta_ref, a_ref, out_ref,
                      ht_scratch):
    @pl.when(pl.program_id(0) == 0)
    def _compute_ht():
        ht = jnp.dot(hf_ref[...], w_ref[...],
                     preferred_element_type=jnp.float32)
        mean = jnp.mean(ht, axis=0, keepdims=True)
        var = jnp.mean(jnp.square(ht - mean), axis=0, keepdims=True)
        inv = jax.lax.rsqrt(var + _BN_EPS)
        htn = (ht - mean) * inv * gamma_ref[...] + beta_ref[...]
        ht_scratch[...] = jnp.maximum(htn, 0.0)

    out_ref[...] = jnp.dot(a_ref[...], ht_scratch[...],
                           preferred_element_type=jnp.float32)


@functools.partial(jax.jit, static_argnames=("block_m", "interpret"))
def _gcn(H, A_normalized, W, bn_gamma, bn_beta, block_m=400, interpret=False):
    batch, n, in_dim = H.shape
    out_dim = W.shape[1]
    hf = H.reshape(batch * n, in_dim)
    gamma = bn_gamma.reshape(1, out_dim)
    beta = bn_beta.reshape(1, out_dim)

    grid = (n // block_m,)
    out = pl.pallas_call(
        _fused_gcn_kernel,
        grid=grid,
        in_specs=[
            pl.BlockSpec((batch * n, in_dim), lambda i: (0, 0)),
            pl.BlockSpec((in_dim, out_dim), lambda i: (0, 0)),
            pl.BlockSpec((1, out_dim), lambda i: (0, 0)),
            pl.BlockSpec((1, out_dim), lambda i: (0, 0)),
            pl.BlockSpec((block_m, n), lambda i: (i, 0)),
        ],
        out_specs=pl.BlockSpec((block_m, out_dim), lambda i: (i, 0)),
        out_shape=jax.ShapeDtypeStruct((n, out_dim), jnp.float32),
        scratch_shapes=[pltpu.VMEM((batch * n, out_dim), jnp.float32)],
        compiler_params=pltpu.CompilerParams(
            vmem_limit_bytes=64 * 1024 * 1024),
        interpret=interpret,
    )(hf, W, gamma, beta, A_normalized)
    return out.reshape(batch, n, out_dim)


def kernel(H, A_normalized, W, bn_gamma, bn_beta):
    import kernel_manual
    return kernel_manual._gcn_manual(H, A_normalized, W, bn_gamma, bn_beta,
                                     block_m=200, nbuf=4)


# speedup vs baseline: 1.0133x; 1.0017x over previous
import functools

import jax
import jax.numpy as jnp
from jax.experimental import pallas as pl
from jax.experimental.pallas import tpu as pltpu

_BN_EPS = 1e-5


def _fused_gcn_kernel(hf_ref, w_ref, gamma_ref, beta_ref, a_ref, out_ref,
                      ht_scratch):
    @pl.when(pl.program_id(0) == 0)
    def _compute_ht():
        ht = jnp.dot(hf_ref[...], w_ref[...],
                     preferred_element_type=jnp.float32)
        mean = jnp.mean(ht, axis=0, keepdims=True)
        var = jnp.mean(jnp.square(ht - mean), axis=0, keepdims=True)
        inv = jax.lax.rsqrt(var + _BN_EPS)
        htn = (ht - mean) * inv * gamma_ref[...] + beta_ref[...]
        ht_scratch[...] = jnp.maximum(htn, 0.0)

    out_ref[...] = jnp.dot(a_ref[...], ht_scratch[...],
                           preferred_element_type=jnp.float32)


@functools.partial(jax.jit, static_argnames=("block_m", "interpret"))
def _gcn(H, A_normalized, W, bn_gamma, bn_beta, block_m=400, interpret=False):
    batch, n, in_dim = H.shape
    out_dim = W.shape[1]
    hf = H.reshape(batch * n, in_dim)
    gamma = bn_gamma.reshape(1, out_dim)
    beta = bn_beta.reshape(1, out_dim)

    grid = (n // block_m,)
    out = pl.pallas_call(
        _fused_gcn_kernel,
        grid=grid,
        in_specs=[
            pl.BlockSpec((batch * n, in_dim), lambda i: (0, 0)),
            pl.BlockSpec((in_dim, out_dim), lambda i: (0, 0)),
            pl.BlockSpec((1, out_dim), lambda i: (0, 0)),
            pl.BlockSpec((1, out_dim), lambda i: (0, 0)),
            pl.BlockSpec((block_m, n), lambda i: (i, 0)),
        ],
        out_specs=pl.BlockSpec((block_m, out_dim), lambda i: (i, 0)),
        out_shape=jax.ShapeDtypeStruct((n, out_dim), jnp.float32),
        scratch_shapes=[pltpu.VMEM((batch * n, out_dim), jnp.float32)],
        compiler_params=pltpu.CompilerParams(
            vmem_limit_bytes=64 * 1024 * 1024),
        interpret=interpret,
    )(hf, W, gamma, beta, A_normalized)
    return out.reshape(batch, n, out_dim)


def kernel(H, A_normalized, W, bn_gamma, bn_beta):
    import kernel_manual
    return kernel_manual._gcn_manual(H, A_normalized, W, bn_gamma, bn_beta,
                                     block_m=200, nbuf=6)


# speedup vs baseline: 1.0272x; 1.0138x over previous
import functools

import jax
import jax.numpy as jnp
from jax.experimental import pallas as pl
from jax.experimental.pallas import tpu as pltpu

_BN_EPS = 1e-5


def _fused_gcn_kernel(hf_ref, w_ref, gamma_ref, beta_ref, a_ref, out_ref,
                      ht_scratch):
    @pl.when(pl.program_id(0) == 0)
    def _compute_ht():
        ht = jnp.dot(hf_ref[...], w_ref[...],
                     preferred_element_type=jnp.float32)
        mean = jnp.mean(ht, axis=0, keepdims=True)
        var = jnp.mean(jnp.square(ht - mean), axis=0, keepdims=True)
        inv = jax.lax.rsqrt(var + _BN_EPS)
        htn = (ht - mean) * inv * gamma_ref[...] + beta_ref[...]
        ht_scratch[...] = jnp.maximum(htn, 0.0)

    out_ref[...] = jnp.dot(a_ref[...], ht_scratch[...],
                           preferred_element_type=jnp.float32)


@functools.partial(jax.jit, static_argnames=("block_m", "interpret"))
def _gcn(H, A_normalized, W, bn_gamma, bn_beta, block_m=400, interpret=False):
    batch, n, in_dim = H.shape
    out_dim = W.shape[1]
    hf = H.reshape(batch * n, in_dim)
    gamma = bn_gamma.reshape(1, out_dim)
    beta = bn_beta.reshape(1, out_dim)

    grid = (n // block_m,)
    out = pl.pallas_call(
        _fused_gcn_kernel,
        grid=grid,
        in_specs=[
            pl.BlockSpec((batch * n, in_dim), lambda i: (0, 0)),
            pl.BlockSpec((in_dim, out_dim), lambda i: (0, 0)),
            pl.BlockSpec((1, out_dim), lambda i: (0, 0)),
            pl.BlockSpec((1, out_dim), lambda i: (0, 0)),
            pl.BlockSpec((block_m, n), lambda i: (i, 0)),
        ],
        out_specs=pl.BlockSpec((block_m, out_dim), lambda i: (i, 0)),
        out_shape=jax.ShapeDtypeStruct((n, out_dim), jnp.float32),
        scratch_shapes=[pltpu.VMEM((batch * n, out_dim), jnp.float32)],
        compiler_params=pltpu.CompilerParams(
            vmem_limit_bytes=64 * 1024 * 1024),
        interpret=interpret,
    )(hf, W, gamma, beta, A_normalized)
    return out.reshape(batch, n, out_dim)


def kernel(H, A_normalized, W, bn_gamma, bn_beta):
    import kernel_manual
    return kernel_manual._gcn_manual(H, A_normalized, W, bn_gamma, bn_beta,
                                     block_m=80, nbuf=12)
